# baseline (speedup 1.0000x reference)
import functools

import jax
import jax.numpy as jnp
from jax import lax
from jax.experimental import pallas as pl
from jax.experimental.pallas import tpu as pltpu
from jax.experimental.pallas import tpu_sc as plsc

_BATCH = 16384
_FIELDS = 26
_DIM = 64

_NC = 2
_NS = 16
_NW = _NC * _NS
_RPW = _BATCH // _NW
_NBUF = 8


def _embed_body(tbl_hbm, idx_hbm, out_hbm, idx_v, rows_v, gsem):
    wid = lax.axis_index("s") * _NC + lax.axis_index("c")
    base = wid * _RPW

    pltpu.sync_copy(idx_hbm.at[pl.ds(base, _RPW)], idx_v)

    def start_gather(row, slot):
        pltpu.make_async_copy(
            tbl_hbm.at[idx_v.at[row]], rows_v.at[slot], gsem.at[slot]
        ).start()

    for b in range(_NBUF):
        start_gather(b, b)

    def outer(j0):
        for b in range(_NBUF):
            row = j0 + b
            pltpu.make_async_copy(
                tbl_hbm.at[idx_v.at[row]], rows_v.at[b], gsem.at[b]
            ).wait()
            pltpu.sync_copy(rows_v.at[b], out_hbm.at[base + row])

            @pl.when(row + _NBUF < _RPW)
            def _():
                start_gather(row + _NBUF, b)

    pl.loop(0, _RPW, step=_NBUF)(outer)


@functools.partial(
    pl.kernel,
    mesh=plsc.VectorSubcoreMesh(core_axis_name="c", subcore_axis_name="s"),
    out_type=jax.ShapeDtypeStruct((_BATCH, _FIELDS, _DIM), jnp.float32),
    scratch_types=[
        pltpu.VMEM((_RPW, _FIELDS), jnp.int32),
        pltpu.VMEM((_NBUF, _FIELDS, _DIM), jnp.float32),
        pltpu.SemaphoreType.DMA((_NBUF,)),
    ],
    compiler_params=pltpu.CompilerParams(use_tc_tiling_on_sc=False),
)
def _embed_call(tbl_hbm, idx_hbm, out_hbm, idx_v, rows_v, gsem):
    _embed_body(tbl_hbm, idx_hbm, out_hbm, idx_v, rows_v, gsem)


def kernel(input, weight):
    return _embed_call(weight, input.astype(jnp.int32))


# speedup vs baseline: 1.2321x; 1.2321x over previous
import functools

import jax
import jax.numpy as jnp
from jax import lax
from jax.experimental import pallas as pl
from jax.experimental.pallas import tpu as pltpu
from jax.experimental.pallas import tpu_sc as plsc

_BATCH = 16384
_FIELDS = 26
_DIM = 64
_PAD = 128

_NC = 2
_NS = 16
_NW = _NC * _NS
_BPW = _BATCH // _NW
_CHUNK = 128
_CPF = _BPW // _CHUNK
_NCH = _FIELDS * _CPF
_NBUF = 4


def _embed_body(tbl_hbm, idx_hbm, out_hbm, idx_v, rows_v, gsem):
    wid = lax.axis_index("s") * _NC + lax.axis_index("c")
    base = wid * _BPW

    pltpu.sync_copy(idx_hbm.at[:, pl.ds(base, _BPW)], idx_v)

    def start_gather(k, slot):
        f = k // _CPF
        c = lax.rem(k, _CPF)
        pltpu.make_async_copy(
            tbl_hbm.at[idx_v.at[f, pl.ds(c * _CHUNK, _CHUNK)]],
            rows_v.at[slot],
            gsem.at[slot],
        ).start()

    for b in range(_NBUF):
        start_gather(b, b)

    def outer(k0):
        for b in range(_NBUF):
            k = k0 + b
            f = k // _CPF
            c = lax.rem(k, _CPF)
            pltpu.make_async_copy(
                tbl_hbm.at[idx_v.at[f, pl.ds(c * _CHUNK, _CHUNK)]],
                rows_v.at[b],
                gsem.at[b],
            ).wait()
            pltpu.sync_copy(
                rows_v.at[b],
                out_hbm.at[pl.ds(base + c * _CHUNK, _CHUNK), f],
            )

            @pl.when(k + _NBUF < _NCH)
            def _():
                start_gather(k + _NBUF, b)

    pl.loop(0, _NCH, step=_NBUF)(outer)


@functools.partial(
    pl.kernel,
    mesh=plsc.VectorSubcoreMesh(core_axis_name="c", subcore_axis_name="s"),
    out_type=jax.ShapeDtypeStruct((_BATCH, _FIELDS, _PAD), jnp.float32),
    scratch_types=[
        pltpu.VMEM((_FIELDS, _BPW), jnp.int32),
        pltpu.VMEM((_NBUF, _CHUNK, _PAD), jnp.float32),
        pltpu.SemaphoreType.DMA((_NBUF,)),
    ],
    compiler_params=pltpu.CompilerParams(use_tc_tiling_on_sc=True),
)
def _embed_call(tbl_hbm, idx_hbm, out_hbm, idx_v, rows_v, gsem):
    _embed_body(tbl_hbm, idx_hbm, out_hbm, idx_v, rows_v, gsem)


def kernel(input, weight):
    wpad = jnp.pad(weight, ((0, 0), (0, _PAD - _DIM)))
    idx_t = input.astype(jnp.int32).T
    return _embed_call(wpad, idx_t)[:, :, :_DIM]
